# Initial kernel scaffold; baseline (speedup 1.0000x reference)
import jax
import jax.numpy as jnp
from jax.experimental import pallas as pl


def kernel(x, indices, weight, delta_weights):
    raise NotImplementedError("write your pallas kernel here")


# speedup vs baseline: 1.3561x; 1.3561x over previous
import functools

import jax
import jax.numpy as jnp
from jax import lax
from jax.experimental import pallas as pl
from jax.experimental.pallas import tpu as pltpu
from jax.experimental.pallas import tpu_sc as plsc

VOCAB = 100000
DIM = 128
MAX_DELTAS = 4
NTOK = 8192

NUM_CORES = 2
NUM_SUBCORES = 16
NW = NUM_CORES * NUM_SUBCORES
TPW = NTOK // NW
CHUNK = 128
NCHUNK = TPW // CHUNK
LANES = 16


def _body(x_hbm, ind_hbm, w_hbm, d_hbm, out_hbm,
          x_v, ind_v, didx_v, a_v, b_v, sem_a, sem_b):
    wid = lax.axis_index("s") * NUM_CORES + lax.axis_index("c")
    base = wid * TPW

    for c in range(NCHUNK):
        off = base + c * CHUNK
        pltpu.sync_copy(x_hbm.at[pl.ds(off, CHUNK)], x_v)
        pltpu.sync_copy(ind_hbm.at[pl.ds(off, CHUNK)], ind_v)

        for j in range(CHUNK // LANES):
            sl = pl.ds(j * LANES, LANES)
            didx_v[sl] = ind_v[sl] * VOCAB + x_v[sl]

        cp_a = pltpu.async_copy(w_hbm.at[x_v], a_v, sem_a)
        cp_b = pltpu.async_copy(d_hbm.at[didx_v], b_v, sem_b)
        cp_a.wait()
        cp_b.wait()

        def add_row(i, _):
            for k in range(DIM // LANES):
                sl = pl.ds(k * LANES, LANES)
                a_v[i, sl] = a_v[i, sl] + b_v[i, sl]
            return _

        lax.fori_loop(0, CHUNK, add_row, 0)

        pltpu.sync_copy(a_v, out_hbm.at[pl.ds(off, CHUNK)])


@jax.jit
def _run(x, indices, weight, dflat):
    mesh = plsc.VectorSubcoreMesh(
        core_axis_name="c", subcore_axis_name="s",
        num_cores=NUM_CORES, num_subcores=NUM_SUBCORES)
    f = pl.kernel(
        _body,
        out_type=jax.ShapeDtypeStruct((NTOK, DIM), jnp.float32),
        mesh=mesh,
        scratch_types=[
            pltpu.VMEM((CHUNK,), jnp.int32),
            pltpu.VMEM((CHUNK,), jnp.int32),
            pltpu.VMEM((CHUNK,), jnp.int32),
            pltpu.VMEM((CHUNK, DIM), jnp.float32),
            pltpu.VMEM((CHUNK, DIM), jnp.float32),
            pltpu.SemaphoreType.DMA,
            pltpu.SemaphoreType.DMA,
        ],
    )
    return f(x, indices, weight, dflat)


def kernel(x, indices, weight, delta_weights):
    dflat = delta_weights.reshape(MAX_DELTAS * VOCAB, DIM)
    return _run(x, indices, weight, dflat)


# speedup vs baseline: 1.3990x; 1.0317x over previous
import functools

import jax
import jax.numpy as jnp
from jax import lax
from jax.experimental import pallas as pl
from jax.experimental.pallas import tpu as pltpu
from jax.experimental.pallas import tpu_sc as plsc

VOCAB = 100000
DIM = 128
MAX_DELTAS = 4
NTOK = 8192

NUM_CORES = 2
NUM_SUBCORES = 16
NW = NUM_CORES * NUM_SUBCORES
TPW = NTOK // NW
CHUNK = 128
NCHUNK = TPW // CHUNK
LANES = 16


def _body(x_hbm, ind_hbm, w_hbm, d_hbm, out_hbm,
          x_v, ind_v, didx_v, a_v, b_v, sem_a, sem_b):
    wid = lax.axis_index("s") * NUM_CORES + lax.axis_index("c")
    base = wid * TPW

    for c in range(NCHUNK):
        off = base + c * CHUNK
        pltpu.sync_copy(x_hbm.at[pl.ds(off, CHUNK)], x_v)
        pltpu.sync_copy(ind_hbm.at[pl.ds(off, CHUNK)], ind_v)

        for j in range(CHUNK // LANES):
            sl = pl.ds(j * LANES, LANES)
            didx_v[sl] = ind_v[sl] * VOCAB + x_v[sl]

        pltpu.async_copy(w_hbm.at[x_v], a_v, sem_a).wait()
        pltpu.async_copy(d_hbm.at[didx_v], a_v, sem_b, add=True).wait()

        pltpu.sync_copy(a_v, out_hbm.at[pl.ds(off, CHUNK)])


@jax.jit
def _run(x, indices, weight, dflat):
    mesh = plsc.VectorSubcoreMesh(
        core_axis_name="c", subcore_axis_name="s",
        num_cores=NUM_CORES, num_subcores=NUM_SUBCORES)
    f = pl.kernel(
        _body,
        out_type=jax.ShapeDtypeStruct((NTOK, DIM), jnp.float32),
        mesh=mesh,
        scratch_types=[
            pltpu.VMEM((CHUNK,), jnp.int32),
            pltpu.VMEM((CHUNK,), jnp.int32),
            pltpu.VMEM((CHUNK,), jnp.int32),
            pltpu.VMEM((CHUNK, DIM), jnp.float32),
            pltpu.VMEM((CHUNK, DIM), jnp.float32),
            pltpu.SemaphoreType.DMA,
            pltpu.SemaphoreType.DMA,
        ],
    )
    return f(x, indices, weight, dflat)


def kernel(x, indices, weight, delta_weights):
    dflat = delta_weights.reshape(MAX_DELTAS * VOCAB, DIM)
    return _run(x, indices, weight, dflat)


# speedup vs baseline: 1.5988x; 1.1428x over previous
import jax
import jax.numpy as jnp
from jax import lax
from jax.experimental import pallas as pl
from jax.experimental.pallas import tpu as pltpu
from jax.experimental.pallas import tpu_sc as plsc

VOCAB = 100000
DIM = 128
MAX_DELTAS = 4
NTOK = 8192

NUM_CORES = 2
NUM_SUBCORES = 16
NW = NUM_CORES * NUM_SUBCORES
TPW = NTOK // NW
CHUNK = 64
NCH = TPW // CHUNK
LANES = 16


def _body(x_hbm, ind_hbm, w_hbm, d_hbm, out_hbm,
          x_v, ind_v, didx_v, buf, sem_i, sem_a, sem_b, sem_o):
    wid = lax.axis_index("s") * NUM_CORES + lax.axis_index("c")
    base = wid * TPW

    cp_x = pltpu.async_copy(x_hbm.at[pl.ds(base, TPW)], x_v, sem_i)
    cp_i = pltpu.async_copy(ind_hbm.at[pl.ds(base, TPW)], ind_v, sem_i)
    cp_x.wait()
    cp_i.wait()

    for j in range(TPW // LANES):
        sl = pl.ds(j * LANES, LANES)
        didx_v[sl] = ind_v[sl] * VOCAB + x_v[sl]

    cps_a = [
        pltpu.async_copy(
            w_hbm.at[x_v.at[pl.ds(c * CHUNK, CHUNK)]], buf.at[c], sem_a)
        for c in range(NCH)
    ]
    cps_b = []
    for c in range(NCH):
        cps_a[c].wait()
        cps_b.append(pltpu.async_copy(
            d_hbm.at[didx_v.at[pl.ds(c * CHUNK, CHUNK)]], buf.at[c], sem_b,
            add=True))
    cps_o = []
    for c in range(NCH):
        cps_b[c].wait()
        cps_o.append(pltpu.async_copy(
            buf.at[c], out_hbm.at[pl.ds(base + c * CHUNK, CHUNK)], sem_o))
    for c in range(NCH):
        cps_o[c].wait()


@jax.jit
def _run(x, indices, weight, dflat):
    mesh = plsc.VectorSubcoreMesh(
        core_axis_name="c", subcore_axis_name="s",
        num_cores=NUM_CORES, num_subcores=NUM_SUBCORES)
    f = pl.kernel(
        _body,
        out_type=jax.ShapeDtypeStruct((NTOK, DIM), jnp.float32),
        mesh=mesh,
        scratch_types=[
            pltpu.VMEM((TPW,), jnp.int32),
            pltpu.VMEM((TPW,), jnp.int32),
            pltpu.VMEM((TPW,), jnp.int32),
            pltpu.VMEM((NCH, CHUNK, DIM), jnp.float32),
            pltpu.SemaphoreType.DMA,
            pltpu.SemaphoreType.DMA,
            pltpu.SemaphoreType.DMA,
            pltpu.SemaphoreType.DMA,
        ],
    )
    return f(x, indices, weight, dflat)


def kernel(x, indices, weight, delta_weights):
    dflat = delta_weights.reshape(MAX_DELTAS * VOCAB, DIM)
    return _run(x, indices, weight, dflat)
